# baseline (speedup 1.0000x reference)
import jax
import jax.numpy as jnp
from jax.experimental import pallas as pl
from jax.experimental.pallas import tpu as pltpu

_NCHUNK = 8


def _copy_body(src_ref, dst_ref, sems):
    rows = src_ref.shape[0] // _NCHUNK
    copies = []
    for i in range(_NCHUNK):
        c = pltpu.make_async_copy(
            src_ref.at[pl.ds(i * rows, rows), :],
            dst_ref.at[pl.ds(i * rows, rows), :],
            sems.at[i],
        )
        c.start()
        copies.append(c)
    for c in copies:
        c.wait()


def kernel(med_embeddings):
    n, d = med_embeddings.shape
    wide = med_embeddings.reshape(n * d // 128, 128)
    out = pl.pallas_call(
        _copy_body,
        in_specs=[pl.BlockSpec(memory_space=pltpu.MemorySpace.HBM)],
        out_specs=pl.BlockSpec(memory_space=pltpu.MemorySpace.HBM),
        out_shape=jax.ShapeDtypeStruct(wide.shape, wide.dtype),
        scratch_shapes=[pltpu.SemaphoreType.DMA((_NCHUNK,))],
    )(wide)
    return out.reshape(n, d)


# speedup vs baseline: 3.0159x; 3.0159x over previous
import jax
import jax.numpy as jnp
from jax.experimental import pallas as pl
from jax.experimental.pallas import tpu as pltpu

_BLOCK = 25_000


def _copy_body(src_ref, dst_ref):
    dst_ref[...] = src_ref[...]


def kernel(med_embeddings):
    n, d = med_embeddings.shape
    wide = med_embeddings.reshape(n * d // 128, 128)
    out = pl.pallas_call(
        _copy_body,
        grid=(wide.shape[0] // _BLOCK,),
        in_specs=[pl.BlockSpec((_BLOCK, 128), lambda i: (i, 0))],
        out_specs=pl.BlockSpec((_BLOCK, 128), lambda i: (i, 0)),
        out_shape=jax.ShapeDtypeStruct(wide.shape, wide.dtype),
    )(wide)
    return out.reshape(n, d)


# speedup vs baseline: 3.3830x; 1.1217x over previous
import jax
import jax.numpy as jnp
from jax.experimental import pallas as pl
from jax.experimental.pallas import tpu as pltpu

_BLOCK = 20_000


def _copy_body(src_ref, dst_ref):
    dst_ref[...] = src_ref[...]


def kernel(med_embeddings):
    n, d = med_embeddings.shape
    return pl.pallas_call(
        _copy_body,
        grid=(n // _BLOCK,),
        in_specs=[pl.BlockSpec((_BLOCK, d), lambda i: (i, 0))],
        out_specs=pl.BlockSpec((_BLOCK, d), lambda i: (i, 0)),
        out_shape=jax.ShapeDtypeStruct(med_embeddings.shape, med_embeddings.dtype),
    )(med_embeddings)
